# baseline (speedup 1.0000x reference)
import jax
import jax.numpy as jnp
from jax.experimental import pallas as pl

_HEADS = 4
_GAT_HIDDEN = [32, 64, 128]
_ATTN_ORDER = ['atom', 'bond', 'global']
_ATTN_MECH = {'atom': (['bond', 'global'], ['b2a', 'g2a']),
              'bond': (['atom', 'global'], ['a2b', 'g2b']),
              'global': (['atom', 'bond'], ['a2g', 'b2g'])}
_SLOPE = 0.2


def _mm_kernel(x_ref, w_ref, o_ref):
    o_ref[...] = jnp.dot(x_ref[...], w_ref[...],
                         preferred_element_type=jnp.float32)


def _pmm(x, w):
    m, k = x.shape
    n = w.shape[1]
    tm = m if m <= 2048 else 2000
    assert m % tm == 0
    return pl.pallas_call(
        _mm_kernel,
        grid=(m // tm,),
        in_specs=[pl.BlockSpec((tm, k), lambda i: (i, 0)),
                  pl.BlockSpec((k, n), lambda i: (0, 0))],
        out_specs=pl.BlockSpec((tm, n), lambda i: (i, 0)),
        out_shape=jax.ShapeDtypeStruct((m, n), jnp.float32),
    )(x, w)


def _elu(x):
    return jnp.where(x > 0, x, jnp.exp(jnp.minimum(x, 0.0)) - 1.0)


def _fc_kernel(x_ref, w0, b0, w1, b1, w2, b2, w3, b3, o_ref):
    x = x_ref[...]
    x = _elu(jnp.dot(x, w0[...], preferred_element_type=jnp.float32)
             + b0[...])
    x = _elu(jnp.dot(x, w1[...], preferred_element_type=jnp.float32)
             + b1[...])
    x = _elu(jnp.dot(x, w2[...], preferred_element_type=jnp.float32)
             + b2[...])
    o_ref[...] = jnp.dot(x, w3[...], preferred_element_type=jnp.float32) \
        + b3[...]


def _fc_readout(hb, fc_params):
    m, k = hb.shape
    tm = 2000
    assert m % tm == 0
    ws = [p['w'] for p in fc_params]
    bs = [p['b'].reshape(1, -1) for p in fc_params]
    wlast = jnp.zeros((ws[3].shape[0], 128), jnp.float32).at[:, :1].set(ws[3])
    blast = jnp.zeros((1, 128), jnp.float32).at[:, :1].set(bs[3])
    specs = [pl.BlockSpec((tm, k), lambda i: (i, 0))]
    for w, b in zip(ws[:3] + [wlast], bs[:3] + [blast]):
        specs.append(pl.BlockSpec(w.shape, lambda i: (0, 0)))
        specs.append(pl.BlockSpec(b.shape, lambda i: (0, 0)))
    out = pl.pallas_call(
        _fc_kernel,
        grid=(m // tm,),
        in_specs=specs,
        out_specs=pl.BlockSpec((tm, 128), lambda i: (i, 0)),
        out_shape=jax.ShapeDtypeStruct((m, 128), jnp.float32),
    )(hb, ws[0], bs[0], ws[1], bs[1], ws[2], bs[2], wlast, blast)
    return out[:, 0]


def _segment_softmax(logits, seg, num):
    m = jax.ops.segment_max(logits, seg, num_segments=num)
    m = jnp.where(jnp.isfinite(m), m, 0.0)
    e = jnp.exp(logits - m[seg])
    s = jax.ops.segment_sum(e, seg, num_segments=num)
    return e / (s[seg] + 1e-9)


def kernel(feat_atom, feat_bond, feat_global, bond_atoms, atom_graph,
           bond_graph, params):
    na = feat_atom.shape[0]
    nb = feat_bond.shape[0]
    bidx = jnp.repeat(jnp.arange(nb), 2)
    aflat = bond_atoms.reshape(-1)
    edges = {'a2b': (aflat, bidx),
             'b2a': (bidx, aflat),
             'g2a': (atom_graph, jnp.arange(na)),
             'a2g': (jnp.arange(na), atom_graph),
             'g2b': (bond_graph, jnp.arange(nb)),
             'b2g': (jnp.arange(nb), bond_graph)}
    nnodes = {'atom': na, 'bond': nb, 'global': feat_global.shape[0]}
    h = {'atom': feat_atom, 'bond': feat_bond, 'global': feat_global}
    for i, out in enumerate(_GAT_HIDDEN):
        layer = params['layers'][i]
        for t in _ATTN_ORDER:
            p = layer[t]
            neighs, etypes = _ATTN_MECH[t]
            nt = nnodes[t]
            ft_dst = _pmm(h[t], p['fc_dst']).reshape(nt, _HEADS, out)
            el = jnp.sum(ft_dst * p['attn_l'][None], axis=-1)
            logit_list, msg_list, dst_list = [], [], []
            for n, e in zip(neighs, etypes):
                src, dst = edges[e]
                nn_ = nnodes[n]
                ft_src = _pmm(h[n], p['fc_src'][n]).reshape(nn_, _HEADS, out)
                er = jnp.sum(ft_src * p['attn_r'][n][None], axis=-1)
                logit_list.append(el[dst] + er[src])
                msg_list.append(ft_src[src])
                dst_list.append(dst)
            logits = jax.nn.leaky_relu(jnp.concatenate(logit_list, 0), _SLOPE)
            msgs = jnp.concatenate(msg_list, 0)
            dsts = jnp.concatenate(dst_list, 0)
            a = _segment_softmax(logits, dsts, nt)
            rst = jax.ops.segment_sum(msgs * a[:, :, None], dsts,
                                      num_segments=nt)
            if i > 0:
                rst = rst + _pmm(h[t], p['res_fc']).reshape(nt, _HEADS, out)
            h[t] = jax.nn.elu(rst).reshape(nt, _HEADS * out)
    src, dst = edges['a2b']
    gathered = h['atom'][src]
    ssum = jax.ops.segment_sum(gathered, dst, num_segments=nb)
    cnt = jax.ops.segment_sum(jnp.ones((src.shape[0],), jnp.float32), dst,
                              num_segments=nb)
    mean = ssum / jnp.maximum(cnt, 1.0)[:, None]
    smax = jax.ops.segment_max(gathered, dst, num_segments=nb)
    smax = jnp.where(jnp.isfinite(smax), smax, 0.0)
    hb = jnp.concatenate([h['bond'], mean, smax], axis=1)
    return _fc_readout(hb, params['fc'])
